# baseline (speedup 1.0000x reference)
import functools
import math

import jax
import jax.numpy as jnp
from jax import lax
from jax.experimental import pallas as pl
from jax.experimental.pallas import tpu as pltpu
from jax.experimental.pallas import tpu_sc as plsc

B_I = 4096
B_J = 200
DIM = 64
LANES = 16
NCORES = 2
NSUB = 16
NW = NCORES * NSUB
CHUNK = B_I // NW
NBUF = 4
SCALE = math.sqrt(DIM)


def _emb_body(idx_hbm, table_hbm, out_hbm, idx_v, *scratch):
    grows = scratch[0:NBUF]
    trows = scratch[NBUF:2 * NBUF]
    gsems = scratch[2 * NBUF:3 * NBUF]
    osems = scratch[3 * NBUF:4 * NBUF]

    wid = lax.axis_index("s") * NCORES + lax.axis_index("c")
    ibase = wid * CHUNK

    pltpu.sync_copy(idx_hbm.at[:, pl.ds(ibase, CHUNK)], idx_v)

    def start_gather(j, b):
        pltpu.async_copy(table_hbm.at[idx_v.at[j]], grows[b], gsems[b])

    def wait_gather(j, b):
        pltpu.make_async_copy(table_hbm.at[idx_v.at[j]], grows[b], gsems[b]).wait()

    def out_slice(j):
        return out_hbm.at[j, :, pl.ds(ibase, CHUNK)]

    def start_store(j, b):
        pltpu.async_copy(trows[b], out_slice(j), osems[b])

    def wait_store(j, b):
        pltpu.make_async_copy(trows[b], out_slice(j), osems[b]).wait()

    riota = [lax.iota(jnp.int32, LANES) + r0 for r0 in range(0, CHUNK, LANES)]

    def transpose_scale(b):
        def col(c, carry):
            cvec = jnp.full((LANES,), c, dtype=jnp.int32)
            for k in range(CHUNK // LANES):
                vals = plsc.load_gather(grows[b], [riota[k], cvec])
                trows[b][c, pl.ds(k * LANES, LANES)] = vals * SCALE
            return carry
        lax.fori_loop(0, DIM, col, 0, unroll=2)

    for b in range(NBUF):
        start_gather(b, b)
    for b in range(NBUF):
        wait_gather(b, b)
        transpose_scale(b)
        start_store(b, b)
        start_gather(b + NBUF, b)

    def group(k, carry):
        j0 = k * NBUF
        for b in range(NBUF):
            j = j0 + b
            wait_gather(j, b)
            wait_store(j - NBUF, b)
            transpose_scale(b)
            start_store(j, b)
            start_gather(j + NBUF, b)
        return carry
    lax.fori_loop(1, B_J // NBUF - 1, group, 0)

    for b in range(NBUF):
        j = B_J - NBUF + b
        wait_gather(j, b)
        wait_store(j - NBUF, b)
        transpose_scale(b)
        start_store(j, b)
    for b in range(NBUF):
        wait_store(B_J - NBUF + b, b)


_emb = functools.partial(
    pl.kernel,
    mesh=plsc.VectorSubcoreMesh(core_axis_name="c", subcore_axis_name="s"),
    out_type=jax.ShapeDtypeStruct((B_J, DIM, B_I), jnp.float32),
    compiler_params=pltpu.CompilerParams(
        use_tc_tiling_on_sc=False, needs_layout_passes=False
    ),
    scratch_types=(
        [pltpu.VMEM((B_J, CHUNK), jnp.int32)]
        + [pltpu.VMEM((CHUNK, DIM), jnp.float32)] * NBUF
        + [pltpu.VMEM((DIM, CHUNK), jnp.float32)] * NBUF
        + [pltpu.SemaphoreType.DMA] * (2 * NBUF)
    ),
)(_emb_body)


def kernel(x, table):
    xt = jnp.transpose(x)
    out_t = _emb(xt, table)
    return jnp.transpose(out_t, (2, 0, 1))
